# baseline (device time: 56920 ns/iter reference)
import jax
import jax.numpy as jnp
from jax import lax
from jax.experimental import pallas as pl
from jax.experimental.pallas import tpu as pltpu

N_DEV = 16
B, SQ, DM = 2, 512, 768
DH = 64
BLK = 64
ROWS = B * SQ
CH = ROWS // N_DEV


def kernel(x, Wq, K_ext, V_ext, Wo):
    H = K_ext.shape[2]
    HD = H * DH

    idx = lax.axis_index("i")
    x2 = x.reshape(ROWS, DM).astype(jnp.bfloat16)
    wq_s = lax.dynamic_slice(Wq, (0, idx * HD), (DM, HD)).astype(jnp.bfloat16)
    wo_s = lax.dynamic_slice(Wo, (idx * HD, 0), (HD, DM)).astype(jnp.bfloat16)
    k = jnp.transpose(K_ext, (0, 2, 1, 3)).astype(jnp.bfloat16)
    v = jnp.transpose(V_ext, (0, 2, 1, 3)).astype(jnp.bfloat16)

    def body(x_ref, wq_ref, k_ref, v_ref, wo_ref, o_ref,
             q_ref, ctx_ref, part_ref, red_ref, rs_recv,
             rs_ssem, rs_rsem, ag_ssem, ag_rsem):
        me = lax.axis_index("i")

        q_ref[...] = jnp.dot(
            x_ref[...], wq_ref[...], preferred_element_type=jnp.float32
        ).astype(jnp.bfloat16)

        qb = lax.broadcasted_iota(jnp.int32, (SQ, SQ), 0) // BLK
        kb = lax.broadcasted_iota(jnp.int32, (SQ, SQ), 1) // BLK
        mask = (qb == kb) | (kb == 0) | ((qb + kb) % 3 == 0)
        bias = jnp.where(mask, 0.0, -1e9).astype(jnp.float32)

        for b in range(B):
            for h in range(H):
                q = q_ref[b * SQ:(b + 1) * SQ, h * DH:(h + 1) * DH]
                s = lax.dot_general(
                    q, k_ref[b, h], (((1,), (1,)), ((), ())),
                    preferred_element_type=jnp.float32,
                )
                e = jnp.exp(s * 0.125 + bias)
                rsum = jnp.sum(e, axis=1, keepdims=True)
                ctx = jnp.dot(e.astype(jnp.bfloat16), v_ref[b, h],
                              preferred_element_type=jnp.float32)
                ctx = ctx * (1.0 / rsum)
                ctx_ref[b * SQ:(b + 1) * SQ, h * DH:(h + 1) * DH] = (
                    ctx.astype(jnp.bfloat16))

        bar = pltpu.get_barrier_semaphore()
        for j in range(N_DEV):
            pl.semaphore_signal(bar, inc=1, device_id=(j,),
                                device_id_type=pl.DeviceIdType.MESH)
        pl.semaphore_wait(bar, N_DEV)

        rs_descs = []
        for o in range(N_DEV):
            j = (me + o) % N_DEV
            rows = pl.ds(j * CH, CH)
            pc = jnp.dot(ctx_ref[rows, :], wo_ref[...],
                         preferred_element_type=jnp.float32).astype(jnp.bfloat16)
            part_ref[rows, :] = pc
            if o == 0:
                rs_recv[me, :, :] = pc
            else:
                d = pltpu.make_async_remote_copy(
                    src_ref=part_ref.at[rows, :],
                    dst_ref=rs_recv.at[me],
                    send_sem=rs_ssem.at[o - 1],
                    recv_sem=rs_rsem,
                    device_id=(j,), device_id_type=pl.DeviceIdType.MESH)
                d.start()
                rs_descs.append(d)
        for d in rs_descs:
            d.wait_recv()

        acc = rs_recv[0].astype(jnp.float32)
        for s_ in range(1, N_DEV):
            acc = acc + rs_recv[s_].astype(jnp.float32)
        red_ref[...] = acc.astype(jnp.bfloat16)

        myrows = pl.ds(me * CH, CH)
        o_ref[myrows, :] = red_ref[...]
        ag_descs = []
        for o in range(1, N_DEV):
            j = (me + o) % N_DEV
            d = pltpu.make_async_remote_copy(
                src_ref=red_ref,
                dst_ref=o_ref.at[myrows, :],
                send_sem=ag_ssem.at[o - 1],
                recv_sem=ag_rsem,
                device_id=(j,), device_id_type=pl.DeviceIdType.MESH)
            d.start()
            ag_descs.append(d)
        for d in rs_descs:
            d.wait_send()
        for d in ag_descs:
            d.wait_recv()
        for d in ag_descs:
            d.wait_send()

    out = pl.pallas_call(
        body,
        out_shape=jax.ShapeDtypeStruct((ROWS, DM), jnp.bfloat16),
        in_specs=[pl.BlockSpec(memory_space=pltpu.VMEM)] * 5,
        out_specs=pl.BlockSpec(memory_space=pltpu.VMEM),
        scratch_shapes=[
            pltpu.VMEM((ROWS, HD), jnp.bfloat16),
            pltpu.VMEM((ROWS, HD), jnp.bfloat16),
            pltpu.VMEM((ROWS, DM), jnp.bfloat16),
            pltpu.VMEM((CH, DM), jnp.bfloat16),
            pltpu.VMEM((N_DEV, CH, DM), jnp.bfloat16),
            pltpu.SemaphoreType.DMA((N_DEV - 1,)),
            pltpu.SemaphoreType.DMA,
            pltpu.SemaphoreType.DMA((N_DEV - 1,)),
            pltpu.SemaphoreType.DMA,
        ],
        compiler_params=pltpu.CompilerParams(collective_id=0),
    )(x2, wq_s, k, v, wo_s)

    return out.astype(jnp.float32).reshape(B, SQ, DM)


# device time: 56844 ns/iter; 1.0013x vs baseline; 1.0013x over previous
import os

import jax
import jax.numpy as jnp
from jax import lax
from jax.experimental import pallas as pl
from jax.experimental.pallas import tpu as pltpu

KMODE = os.environ.get("KMODE", "full")

N_DEV = 16
B, SQ, DM = 2, 512, 768
DH = 64
BLK = 64
ROWS = B * SQ
CH = ROWS // N_DEV


def kernel(x, Wq, K_ext, V_ext, Wo):
    H = K_ext.shape[2]
    HD = H * DH

    idx = lax.axis_index("i")
    x2 = x.reshape(ROWS, DM)
    k2 = K_ext.reshape(B, SQ, HD)
    v2 = V_ext.reshape(B, SQ, HD)
    wq_s = lax.dynamic_slice(Wq, (0, idx * HD), (DM, HD)).astype(jnp.bfloat16)
    wo_s = lax.dynamic_slice(Wo, (idx * HD, 0), (HD, DM)).astype(jnp.bfloat16)

    def body(x_ref, wq_ref, k_ref, v_ref, wo_ref, o_ref,
             xb_ref, kb_ref, vb_ref, q_ref, ctx_ref, part_ref, red_ref,
             rs_recv, agbuf,
             rs_ssem, rs_rsem, ag_ssem, ag_rsem):
        me = lax.axis_index("i")

        xb_ref[...] = x_ref[...].astype(jnp.bfloat16)
        kb_ref[...] = k_ref[...].astype(jnp.bfloat16)
        vb_ref[...] = v_ref[...].astype(jnp.bfloat16)

        q_ref[...] = jnp.dot(
            xb_ref[...], wq_ref[...], preferred_element_type=jnp.float32
        ).astype(jnp.bfloat16)

        qb = lax.broadcasted_iota(jnp.int32, (SQ, SQ), 0) // BLK
        kb = lax.broadcasted_iota(jnp.int32, (SQ, SQ), 1) // BLK
        mask = (qb == kb) | (kb == 0) | ((qb + kb) % 3 == 0)
        bias = jnp.where(mask, 0.0, -1e9).astype(jnp.float32)

        for b in range(B):
            for h in range(H):
                if KMODE == "noattn":
                    break
                cols = slice(h * DH, (h + 1) * DH)
                q = q_ref[b * SQ:(b + 1) * SQ, cols]
                s = lax.dot_general(
                    q, kb_ref[b, :, cols], (((1,), (1,)), ((), ())),
                    preferred_element_type=jnp.float32,
                )
                e = jnp.exp(s * 0.125 + bias)
                rsum = jnp.sum(e, axis=1, keepdims=True)
                ctx = jnp.dot(e.astype(jnp.bfloat16), vb_ref[b, :, cols],
                              preferred_element_type=jnp.float32)
                ctx = ctx * (1.0 / rsum)
                ctx_ref[b * SQ:(b + 1) * SQ, cols] = ctx.astype(jnp.bfloat16)

        if KMODE == "nocomm":
            for o in range(N_DEV):
                j = (me + o) % N_DEV
                rows = pl.ds(j * CH, CH)
                pc = jnp.dot(ctx_ref[rows, :], wo_ref[...],
                             preferred_element_type=jnp.float32)
                o_ref[rows, :] = pc
            return

        bar = pltpu.get_barrier_semaphore()
        for j in range(N_DEV):
            pl.semaphore_signal(bar, inc=1, device_id=(j,),
                                device_id_type=pl.DeviceIdType.MESH)
        pl.semaphore_wait(bar, N_DEV)

        rs_descs = []
        for o in range(N_DEV):
            j = (me + o) % N_DEV
            rows = pl.ds(j * CH, CH)
            pc = jnp.dot(ctx_ref[rows, :], wo_ref[...],
                         preferred_element_type=jnp.float32).astype(jnp.bfloat16)
            part_ref[rows, :] = pc
            if o == 0:
                rs_recv[me, :, :] = pc
            else:
                d = pltpu.make_async_remote_copy(
                    src_ref=part_ref.at[rows, :],
                    dst_ref=rs_recv.at[me],
                    send_sem=rs_ssem.at[o - 1],
                    recv_sem=rs_rsem.at[o - 1],
                    device_id=(j,), device_id_type=pl.DeviceIdType.MESH)
                d.start()
                rs_descs.append(d)

        acc = rs_recv[me].astype(jnp.float32)
        for o in range(1, N_DEV):
            rs_descs[o - 1].wait_recv()
            acc = acc + rs_recv[(me - o) % N_DEV].astype(jnp.float32)
        red_ref[...] = acc.astype(jnp.bfloat16)

        myrows = pl.ds(me * CH, CH)
        agbuf[myrows, :] = red_ref[...]
        ag_descs = []
        for o in range(1, N_DEV):
            j = (me + o) % N_DEV
            d = pltpu.make_async_remote_copy(
                src_ref=red_ref,
                dst_ref=agbuf.at[myrows, :],
                send_sem=ag_ssem.at[o - 1],
                recv_sem=ag_rsem,
                device_id=(j,), device_id_type=pl.DeviceIdType.MESH)
            d.start()
            ag_descs.append(d)
        for d in rs_descs:
            d.wait_send()
        for d in ag_descs:
            d.wait_recv()
        o_ref[...] = agbuf[...].astype(jnp.float32)
        for d in ag_descs:
            d.wait_send()

    out = pl.pallas_call(
        body,
        out_shape=jax.ShapeDtypeStruct((ROWS, DM), jnp.float32),
        in_specs=[pl.BlockSpec(memory_space=pltpu.VMEM)] * 5,
        out_specs=pl.BlockSpec(memory_space=pltpu.VMEM),
        scratch_shapes=[
            pltpu.VMEM((ROWS, DM), jnp.bfloat16),
            pltpu.VMEM((B, SQ, HD), jnp.bfloat16),
            pltpu.VMEM((B, SQ, HD), jnp.bfloat16),
            pltpu.VMEM((ROWS, HD), jnp.bfloat16),
            pltpu.VMEM((ROWS, HD), jnp.bfloat16),
            pltpu.VMEM((ROWS, DM), jnp.bfloat16),
            pltpu.VMEM((CH, DM), jnp.bfloat16),
            pltpu.VMEM((N_DEV, CH, DM), jnp.bfloat16),
            pltpu.VMEM((ROWS, DM), jnp.bfloat16),
            pltpu.SemaphoreType.DMA((N_DEV - 1,)),
            pltpu.SemaphoreType.DMA((N_DEV - 1,)),
            pltpu.SemaphoreType.DMA((N_DEV - 1,)),
            pltpu.SemaphoreType.DMA,
        ],
        compiler_params=pltpu.CompilerParams(
            collective_id=None if KMODE == "nocomm" else 0),
    )(x2, wq_s, k2, v2, wo_s)

    return out.reshape(B, SQ, DM)
